# baseline (speedup 1.0000x reference)
import jax
import jax.numpy as jnp
import ml_dtypes
import numpy as np
from jax.experimental import pallas as pl
from jax.experimental.pallas import tpu as pltpu

_SMOOTH = 0.025
_FLOOR = 1e-06
_L = 256
_BB = 32


def _pcen_body(x_ref, tri_ref, dcol_ref, a_ref, d_ref, r_ref, o_ref,
               carry_ref):
    t = pl.program_id(1)

    @pl.when(t == 0)
    def _init():
        carry_ref[...] = x_ref[:, 0, :]

    x = x_ref[...]
    tri = tri_ref[...]
    dcol = dcol_ref[...]

    alpha_c = jnp.minimum(a_ref[...], 1.0)
    oor = 1.0 / jnp.maximum(r_ref[...], 1.0)
    delta = d_ref[...]
    t3 = jnp.exp2(oor * jnp.log2(delta))
    for b in range(x.shape[0]):
        xb = x[b]
        xh = xb.astype(jnp.bfloat16)
        m_b = jax.lax.dot(tri, xh, preferred_element_type=jnp.float32)
        m_b = m_b + dcol * carry_ref[b:b + 1, :]
        carry_ref[b:b + 1, :] = m_b[-1:, :]
        inv_t1 = jnp.exp2((-alpha_c) * jnp.log2(_FLOOR + m_b))
        y = xb * inv_t1 + delta
        o_ref[b] = jnp.exp2(oor * jnp.log2(y)) - t3


def _pcen_call(inputs, tri_h, dcol, a2, d2, r2):
    B, T, C = inputs.shape
    nt = T // _L
    nb = B // _BB

    return pl.pallas_call(
        _pcen_body,
        out_shape=jax.ShapeDtypeStruct((B, T, C), jnp.float32),
        grid=(nb, nt),
        in_specs=[
            pl.BlockSpec((_BB, _L, C), lambda ib, it: (ib, it, 0)),
            pl.BlockSpec((_L, _L), lambda ib, it: (0, 0)),
            pl.BlockSpec((_L, 1), lambda ib, it: (0, 0)),
            pl.BlockSpec((1, C), lambda ib, it: (0, 0)),
            pl.BlockSpec((1, C), lambda ib, it: (0, 0)),
            pl.BlockSpec((1, C), lambda ib, it: (0, 0)),
        ],
        out_specs=pl.BlockSpec((_BB, _L, C), lambda ib, it: (ib, it, 0)),
        scratch_shapes=[pltpu.VMEM((_BB, C), jnp.float32)],
        compiler_params=pltpu.CompilerParams(
            dimension_semantics=("parallel", "arbitrary"),
        ),
        name="pcen_fused",
    )(inputs, tri_h, dcol, a2, d2, r2)


def kernel(inputs, alpha, delta, root):
    B, T, C = inputs.shape

    i = np.arange(_L)
    expo = i[:, None] - i[None, :]
    tri = np.where(expo >= 0, _SMOOTH * (1.0 - _SMOOTH) ** np.maximum(expo, 0),
                   0.0).astype(np.float32)
    tri_h = jnp.asarray(tri.astype(np.dtype(ml_dtypes.bfloat16)))
    dcol = jnp.asarray(((1.0 - _SMOOTH) ** (i + 1)).astype(np.float32)
                       .reshape(_L, 1))

    a2 = alpha.reshape(1, C).astype(jnp.float32)
    d2 = delta.reshape(1, C).astype(jnp.float32)
    r2 = root.reshape(1, C).astype(jnp.float32)

    return _pcen_call(inputs, tri_h, dcol, a2, d2, r2)


# speedup vs baseline: 1.0642x; 1.0642x over previous
import jax
import jax.numpy as jnp
import ml_dtypes
import numpy as np
from jax.experimental import pallas as pl
from jax.experimental.pallas import tpu as pltpu

_SMOOTH = 0.025
_FLOOR = 1e-06
_L = 256
_BB = 64


def _pcen_body(x_ref, tri_ref, dcol_ref, a_ref, d_ref, r_ref, o_ref,
               carry_ref):
    t = pl.program_id(1)

    @pl.when(t == 0)
    def _init():
        carry_ref[...] = x_ref[:, 0, :]

    x = x_ref[...]
    tri = tri_ref[...]
    dcol = dcol_ref[...]

    alpha_c = jnp.minimum(a_ref[...], 1.0)
    oor = 1.0 / jnp.maximum(r_ref[...], 1.0)
    delta = d_ref[...]
    t3 = jnp.exp2(oor * jnp.log2(delta))
    for b in range(x.shape[0]):
        xb = x[b]
        xh = xb.astype(jnp.bfloat16)
        m_b = jax.lax.dot(tri, xh, preferred_element_type=jnp.float32)
        m_b = m_b + dcol * carry_ref[b:b + 1, :]
        carry_ref[b:b + 1, :] = m_b[-1:, :]
        inv_t1 = jnp.exp2((-alpha_c) * jnp.log2(_FLOOR + m_b))
        y = xb * inv_t1 + delta
        o_ref[b] = jnp.exp2(oor * jnp.log2(y)) - t3


def _pcen_call(inputs, tri_h, dcol, a2, d2, r2):
    B, T, C = inputs.shape
    nt = T // _L
    nb = B // _BB

    return pl.pallas_call(
        _pcen_body,
        out_shape=jax.ShapeDtypeStruct((B, T, C), jnp.float32),
        grid=(nb, nt),
        in_specs=[
            pl.BlockSpec((_BB, _L, C), lambda ib, it: (ib, it, 0)),
            pl.BlockSpec((_L, _L), lambda ib, it: (0, 0)),
            pl.BlockSpec((_L, 1), lambda ib, it: (0, 0)),
            pl.BlockSpec((1, C), lambda ib, it: (0, 0)),
            pl.BlockSpec((1, C), lambda ib, it: (0, 0)),
            pl.BlockSpec((1, C), lambda ib, it: (0, 0)),
        ],
        out_specs=pl.BlockSpec((_BB, _L, C), lambda ib, it: (ib, it, 0)),
        scratch_shapes=[pltpu.VMEM((_BB, C), jnp.float32)],
        compiler_params=pltpu.CompilerParams(
            dimension_semantics=("parallel", "arbitrary"),
        ),
        name="pcen_fused",
    )(inputs, tri_h, dcol, a2, d2, r2)


def kernel(inputs, alpha, delta, root):
    B, T, C = inputs.shape

    i = np.arange(_L)
    expo = i[:, None] - i[None, :]
    tri = np.where(expo >= 0, _SMOOTH * (1.0 - _SMOOTH) ** np.maximum(expo, 0),
                   0.0).astype(np.float32)
    tri_h = jnp.asarray(tri.astype(np.dtype(ml_dtypes.bfloat16)))
    dcol = jnp.asarray(((1.0 - _SMOOTH) ** (i + 1)).astype(np.float32)
                       .reshape(_L, 1))

    a2 = alpha.reshape(1, C).astype(jnp.float32)
    d2 = delta.reshape(1, C).astype(jnp.float32)
    r2 = root.reshape(1, C).astype(jnp.float32)

    return _pcen_call(inputs, tri_h, dcol, a2, d2, r2)


# speedup vs baseline: 1.1243x; 1.0565x over previous
import jax
import jax.numpy as jnp
import ml_dtypes
import numpy as np
from jax.experimental import pallas as pl
from jax.experimental.pallas import tpu as pltpu

_SMOOTH = 0.025
_FLOOR = 1e-06
_L = 512
_BB = 32


def _pcen_body(x_ref, tri_ref, dcol_ref, a_ref, d_ref, r_ref, o_ref,
               carry_ref):
    t = pl.program_id(1)

    @pl.when(t == 0)
    def _init():
        carry_ref[...] = x_ref[:, 0, :]

    x = x_ref[...]
    tri = tri_ref[...]
    dcol = dcol_ref[...]

    alpha_c = jnp.minimum(a_ref[...], 1.0)
    oor = 1.0 / jnp.maximum(r_ref[...], 1.0)
    delta = d_ref[...]
    t3 = jnp.exp2(oor * jnp.log2(delta))
    for b in range(x.shape[0]):
        xb = x[b]
        xh = xb.astype(jnp.bfloat16)
        m_b = jax.lax.dot(tri, xh, preferred_element_type=jnp.float32)
        m_b = m_b + dcol * carry_ref[b:b + 1, :]
        carry_ref[b:b + 1, :] = m_b[-1:, :]
        inv_t1 = jnp.exp2((-alpha_c) * jnp.log2(_FLOOR + m_b))
        y = xb * inv_t1 + delta
        o_ref[b] = jnp.exp2(oor * jnp.log2(y)) - t3


def _pcen_call(inputs, tri_h, dcol, a2, d2, r2):
    B, T, C = inputs.shape
    nt = T // _L
    nb = B // _BB

    return pl.pallas_call(
        _pcen_body,
        out_shape=jax.ShapeDtypeStruct((B, T, C), jnp.float32),
        grid=(nb, nt),
        in_specs=[
            pl.BlockSpec((_BB, _L, C), lambda ib, it: (ib, it, 0)),
            pl.BlockSpec((_L, _L), lambda ib, it: (0, 0)),
            pl.BlockSpec((_L, 1), lambda ib, it: (0, 0)),
            pl.BlockSpec((1, C), lambda ib, it: (0, 0)),
            pl.BlockSpec((1, C), lambda ib, it: (0, 0)),
            pl.BlockSpec((1, C), lambda ib, it: (0, 0)),
        ],
        out_specs=pl.BlockSpec((_BB, _L, C), lambda ib, it: (ib, it, 0)),
        scratch_shapes=[pltpu.VMEM((_BB, C), jnp.float32)],
        compiler_params=pltpu.CompilerParams(
            dimension_semantics=("parallel", "arbitrary"),
        ),
        name="pcen_fused",
    )(inputs, tri_h, dcol, a2, d2, r2)


def kernel(inputs, alpha, delta, root):
    B, T, C = inputs.shape

    i = np.arange(_L)
    expo = i[:, None] - i[None, :]
    tri = np.where(expo >= 0, _SMOOTH * (1.0 - _SMOOTH) ** np.maximum(expo, 0),
                   0.0).astype(np.float32)
    tri_h = jnp.asarray(tri.astype(np.dtype(ml_dtypes.bfloat16)))
    dcol = jnp.asarray(((1.0 - _SMOOTH) ** (i + 1)).astype(np.float32)
                       .reshape(_L, 1))

    a2 = alpha.reshape(1, C).astype(jnp.float32)
    d2 = delta.reshape(1, C).astype(jnp.float32)
    r2 = root.reshape(1, C).astype(jnp.float32)

    return _pcen_call(inputs, tri_h, dcol, a2, d2, r2)


# speedup vs baseline: 1.1951x; 1.0629x over previous
import jax
import jax.numpy as jnp
import ml_dtypes
import numpy as np
from jax.experimental import pallas as pl
from jax.experimental.pallas import tpu as pltpu

_SMOOTH = 0.025
_FLOOR = 1e-06
_L = 512
_BB = 32


def _pcen_body(x_ref, tri_ref, dcol_ref, a_ref, d_ref, r_ref, o_ref,
               carry_ref):
    t = pl.program_id(1)

    @pl.when(t == 0)
    def _init():
        carry_ref[...] = x_ref[:, 0, :]

    x = x_ref[...]
    tri = tri_ref[...]
    dcol = dcol_ref[...]

    alpha_c = jnp.minimum(a_ref[...], 1.0)
    oor = 1.0 / jnp.maximum(r_ref[...], 1.0)
    delta = d_ref[...]
    t3 = jnp.exp2(oor * jnp.log2(delta))
    row0 = jax.lax.broadcasted_iota(jnp.int32, (x.shape[1], 1), 0) == 0
    k39 = (1.0 - _SMOOTH) / _SMOOTH
    for b in range(x.shape[0]):
        xb = x[b]
        aug = (xb[0:1, :] + k39 * carry_ref[b:b + 1, :]).astype(jnp.bfloat16)
        xh = jnp.where(row0, aug, xb.astype(jnp.bfloat16))
        m_b = jax.lax.dot(tri, xh, preferred_element_type=jnp.float32)
        carry_ref[b:b + 1, :] = m_b[-1:, :]
        inv_t1 = jnp.exp2((-alpha_c) * jnp.log2(_FLOOR + m_b))
        y = xb * inv_t1 + delta
        o_ref[b] = jnp.exp2(oor * jnp.log2(y)) - t3


def _pcen_call(inputs, tri_h, dcol, a2, d2, r2):
    B, T, C = inputs.shape
    nt = T // _L
    nb = B // _BB

    return pl.pallas_call(
        _pcen_body,
        out_shape=jax.ShapeDtypeStruct((B, T, C), jnp.float32),
        grid=(nb, nt),
        in_specs=[
            pl.BlockSpec((_BB, _L, C), lambda ib, it: (ib, it, 0)),
            pl.BlockSpec((_L, _L), lambda ib, it: (0, 0)),
            pl.BlockSpec((_L, 1), lambda ib, it: (0, 0)),
            pl.BlockSpec((1, C), lambda ib, it: (0, 0)),
            pl.BlockSpec((1, C), lambda ib, it: (0, 0)),
            pl.BlockSpec((1, C), lambda ib, it: (0, 0)),
        ],
        out_specs=pl.BlockSpec((_BB, _L, C), lambda ib, it: (ib, it, 0)),
        scratch_shapes=[pltpu.VMEM((_BB, C), jnp.float32)],
        compiler_params=pltpu.CompilerParams(
            dimension_semantics=("parallel", "arbitrary"),
        ),
        name="pcen_fused",
    )(inputs, tri_h, dcol, a2, d2, r2)


def kernel(inputs, alpha, delta, root):
    B, T, C = inputs.shape

    i = np.arange(_L)
    expo = i[:, None] - i[None, :]
    tri = np.where(expo >= 0, _SMOOTH * (1.0 - _SMOOTH) ** np.maximum(expo, 0),
                   0.0).astype(np.float32)
    tri_h = jnp.asarray(tri.astype(np.dtype(ml_dtypes.bfloat16)))
    dcol = jnp.asarray(((1.0 - _SMOOTH) ** (i + 1)).astype(np.float32)
                       .reshape(_L, 1))

    a2 = alpha.reshape(1, C).astype(jnp.float32)
    d2 = delta.reshape(1, C).astype(jnp.float32)
    r2 = root.reshape(1, C).astype(jnp.float32)

    return _pcen_call(inputs, tri_h, dcol, a2, d2, r2)
